# Initial kernel scaffold; baseline (speedup 1.0000x reference)
import jax
import jax.numpy as jnp
from jax import lax
from jax.experimental import pallas as pl
from jax.experimental.pallas import tpu as pltpu
from jax.experimental.pallas import tpu_sc as plsc

NB = 8
N = 20000
NV = N // 16
K = 100
A = 7
KPAD = 112
PROW = 704
SROW = 112

SIGN = jnp.int32(-2147483648)


def _body(scores_hbm, boxes_hbm, prop_hbm, pscore_hbm,
          sbuf, hist, ckey, cidx, cscore, sidx, sscore, eidx, colbuf,
          pbuf, sem):
    wid = lax.axis_index("s") * 2 + lax.axis_index("c")

    @pl.when(wid < NB)
    def _():
        b = wid
        pltpu.sync_copy(scores_hbm.at[b], sbuf)
        lanes = lax.iota(jnp.int32, 16)
        ones = jnp.ones((16,), jnp.int32)

        def keys_at(i):
            v = sbuf[pl.ds(i * 16, 16)]
            bits = plsc.bitcast(v, jnp.int32)
            sg = lax.shift_right_arithmetic(bits, 31)
            ku = bits ^ (sg | SIGN)
            return v, ku

        pv = jnp.int32(0)
        need = jnp.int32(K)
        for p in range(1, 5):
            shift = 32 - 8 * p

            def zero_hist(j, _):
                hist[pl.ds(j * 16, 16)] = jnp.zeros((16,), jnp.int32)
                return 0
            lax.fori_loop(0, 256, zero_hist, 0)

            def scan(i, _, _shift=shift, _p=p, _pv=pv):
                _, ku = keys_at(i)
                digit = lax.shift_right_logical(ku, _shift) & 255
                if _p == 1:
                    mask = None
                else:
                    mask = lax.shift_right_logical(ku, _shift + 8) == _pv
                plsc.addupdate_scatter(hist, [digit * 16 + lanes], ones,
                                       mask=mask)
                return 0
            lax.fori_loop(0, NV, scan, 0)

            def binscan(t, carry):
                found, nd, digit = carry
                dd = 255 - t
                row = hist[pl.ds(dd * 16, 16)]
                c = jnp.sum(row)
                take = jnp.logical_and(jnp.logical_not(found), c >= nd)
                digit = jnp.where(take, dd, digit)
                skip = jnp.logical_or(found, take)
                nd = jnp.where(skip, nd, nd - c)
                return (jnp.logical_or(found, take), nd, digit)

            found, need, digit = lax.fori_loop(
                0, 256, binscan, (jnp.bool_(False), need, jnp.int32(0)))
            pv = pv * 256 + digit

        t_key = pv ^ SIGN
        quota_eq = need

        def zcand(j, _):
            ckey[pl.ds(j * 16, 16)] = jnp.full((16,), SIGN, jnp.int32)
            return 0
        lax.fori_loop(0, 8, zcand, 0)

        def zsidx(j, _):
            sidx[pl.ds(j * 16, 16)] = jnp.zeros((16,), jnp.int32)
            sscore[pl.ds(j * 16, 16)] = jnp.zeros((16,), jnp.float32)
            return 0
        lax.fori_loop(0, 7, zsidx, 0)

        def comp(i, carry):
            nsel, eq_taken = carry
            v, ku = keys_at(i)
            ks = ku ^ SIGN
            m_gt = ks > t_key
            m_eq = ks == t_key
            eq_pref = plsc.cumsum(m_eq.astype(jnp.int32))
            take_eq = jnp.logical_and(m_eq, (eq_taken + eq_pref) <= quota_eq)
            m = jnp.logical_or(m_gt, take_eq)
            idxv = i * 16 + lanes
            plsc.store_compressed(ckey.at[pl.ds(nsel, 16)], ks, mask=m)
            plsc.store_compressed(cidx.at[pl.ds(nsel, 16)], idxv, mask=m)
            plsc.store_compressed(cscore.at[pl.ds(nsel, 16)], v, mask=m)
            cnt = jnp.sum(m.astype(jnp.int32))
            neq = jnp.sum(take_eq.astype(jnp.int32))
            return (nsel + cnt, eq_taken + neq)
        lax.fori_loop(0, NV, comp, (jnp.int32(0), jnp.int32(0)))

        def rankloop(e, _):
            ke = ckey[e]
            cnt = jnp.int32(0)
            for j in range(7):
                kj = ckey[pl.ds(j * 16, 16)]
                pj = j * 16 + lanes
                gt = kj > ke
                eq = jnp.logical_and(kj == ke, pj < e)
                cnt = cnt + jnp.sum(jnp.logical_or(gt, eq).astype(jnp.int32))
            sscore[cnt] = cscore[e]
            sidx[cnt] = cidx[e]
            return 0
        lax.fori_loop(0, K, rankloop, 0)

        base = b * (A * N)
        for a in range(A):
            def mke(j, _, _a=a):
                sv = sidx[pl.ds(j * 16, 16)]
                eidx[pl.ds(j * 16, 16)] = sv + (base + _a * N)
                return 0
            lax.fori_loop(0, 7, mke, 0)
            pltpu.async_copy(boxes_hbm.at[eidx], colbuf, sem).wait()
            for j in range(7):
                v = colbuf[pl.ds(j * 16, 16)]
                pos = (j * 16 + lanes) * A + a
                plsc.store_scatter(pbuf, [pos], v, mask=pos < K * A)

        pltpu.sync_copy(pbuf, prop_hbm.at[b])
        pltpu.sync_copy(sscore, pscore_hbm.at[b])


_MESH = plsc.VectorSubcoreMesh(core_axis_name="c", subcore_axis_name="s",
                               num_cores=2, num_subcores=16)

_SCRATCH = [
    pltpu.VMEM((N,), jnp.float32),
    pltpu.VMEM((4096,), jnp.int32),
    pltpu.VMEM((128,), jnp.int32),
    pltpu.VMEM((128,), jnp.int32),
    pltpu.VMEM((128,), jnp.float32),
    pltpu.VMEM((KPAD,), jnp.int32),
    pltpu.VMEM((SROW,), jnp.float32),
    pltpu.VMEM((KPAD,), jnp.int32),
    pltpu.VMEM((KPAD,), jnp.float32),
    pltpu.VMEM((PROW,), jnp.float32),
    pltpu.SemaphoreType.DMA,
]

_OUT = (jax.ShapeDtypeStruct((NB, PROW), jnp.float32),
        jax.ShapeDtypeStruct((NB, SROW), jnp.float32))

_topk_call = pl.kernel(_body, out_type=_OUT, mesh=_MESH,
                       scratch_types=_SCRATCH)


@jax.jit
def kernel(scores, boxes):
    scores2d = scores.reshape(NB, -1)
    boxes1d = boxes.reshape(-1)
    prop_pad, ps_pad = _topk_call(scores2d, boxes1d)
    proposals = prop_pad[:, :K * A].reshape(NB, K, A)
    pscores = ps_pad[:, :K]
    return (proposals, pscores)


# speedup vs baseline: 1.9701x; 1.9701x over previous
import jax
import jax.numpy as jnp
from jax import lax
from jax.experimental import pallas as pl
from jax.experimental.pallas import tpu as pltpu
from jax.experimental.pallas import tpu_sc as plsc

NB = 8
N = 20000
NV = N // 16
K = 100
A = 7
KPAD = 112
PROW = 704
SROW = 112

SIGN = -2147483648


def _body(scores_hbm, boxes_hbm, prop_hbm, pscore_hbm,
          sbuf, hist, ckey, cidx, cscore, sidx, sscore, eidx, colbuf,
          pbuf, sem):
    wid = lax.axis_index("s") * 2 + lax.axis_index("c")

    @pl.when(wid < NB)
    def _():
        b = wid
        pltpu.sync_copy(scores_hbm.at[b], sbuf)
        lanes = lax.iota(jnp.int32, 16)
        ones = jnp.ones((16,), jnp.int32)

        def keys_at(i):
            v = sbuf[pl.ds(i * 16, 16)]
            bits = lax.bitcast_convert_type(v, jnp.int32)
            sg = lax.shift_right_arithmetic(bits, 31)
            ku = bits ^ (sg | SIGN)
            return v, ku

        pv = jnp.int32(0)
        need = jnp.int32(K)
        for p in range(1, 5):
            shift = 32 - 8 * p

            def zero_hist(j, _):
                hist[pl.ds(j * 16, 16)] = jnp.zeros((16,), jnp.int32)
                return 0
            lax.fori_loop(0, 256, zero_hist, 0)

            def scan(i, _, _shift=shift, _p=p, _pv=pv):
                _, ku = keys_at(i)
                digit = lax.shift_right_logical(ku, _shift) & 255
                if _p == 1:
                    mask = jnp.ones((16,), jnp.bool_)
                else:
                    mask = lax.shift_right_logical(ku, _shift + 8) == _pv
                plsc.addupdate_scatter(hist, [digit * 16 + lanes], ones,
                                       mask=mask)
                return 0
            lax.fori_loop(0, NV, scan, 0)

            def binscan(t, carry):
                found, nd, digit = carry
                dd = 255 - t
                row = hist[pl.ds(dd * 16, 16)]
                c = jnp.sum(row)
                take = jnp.logical_and(jnp.logical_not(found), c >= nd)
                digit = jnp.where(take, dd, digit)
                skip = jnp.logical_or(found, take)
                nd = jnp.where(skip, nd, nd - c)
                return (jnp.logical_or(found, take), nd, digit)

            found, need, digit = lax.fori_loop(
                0, 256, binscan, (jnp.bool_(False), need, jnp.int32(0)))
            pv = pv * 256 + digit

        t_key = pv ^ SIGN
        quota_eq = need

        def zcand(j, _):
            ckey[pl.ds(j * 16, 16)] = jnp.full((16,), SIGN, jnp.int32)
            return 0
        lax.fori_loop(0, 8, zcand, 0)

        def zsidx(j, _):
            sidx[pl.ds(j * 16, 16)] = jnp.zeros((16,), jnp.int32)
            sscore[pl.ds(j * 16, 16)] = jnp.zeros((16,), jnp.float32)
            return 0
        lax.fori_loop(0, 7, zsidx, 0)

        def comp(i, carry):
            nsel, eq_taken = carry
            v, ku = keys_at(i)
            ks = ku ^ SIGN
            m_gt = ks > t_key
            m_eq = ks == t_key
            eq_pref = plsc.cumsum(m_eq.astype(jnp.int32))
            take_eq = jnp.logical_and(m_eq, (eq_taken + eq_pref) <= quota_eq)
            m = jnp.logical_or(m_gt, take_eq)
            idxv = i * 16 + lanes
            plsc.store_compressed(ckey.at[pl.ds(nsel, 16)], ks, mask=m)
            plsc.store_compressed(cidx.at[pl.ds(nsel, 16)], idxv, mask=m)
            plsc.store_compressed(cscore.at[pl.ds(nsel, 16)], v, mask=m)
            cnt = jnp.sum(m.astype(jnp.int32))
            neq = jnp.sum(take_eq.astype(jnp.int32))
            return (nsel + cnt, eq_taken + neq)
        lax.fori_loop(0, NV, comp, (jnp.int32(0), jnp.int32(0)))

        lane0 = lanes == 0

        def rankloop(e, _):
            ke = ckey[pl.ds(e, 16)][0]
            sv = cscore[pl.ds(e, 16)]
            iv = cidx[pl.ds(e, 16)]
            cnt = jnp.int32(0)
            for j in range(7):
                kj = ckey[pl.ds(j * 16, 16)]
                pj = j * 16 + lanes
                gt = kj > ke
                eq = jnp.logical_and(kj == ke, pj < e)
                cnt = cnt + jnp.sum(jnp.logical_or(gt, eq).astype(jnp.int32))
            rankv = jnp.zeros((16,), jnp.int32) + cnt
            plsc.store_scatter(sscore, [rankv], sv, mask=lane0)
            plsc.store_scatter(sidx, [rankv], iv, mask=lane0)
            return 0
        lax.fori_loop(0, K, rankloop, 0)

        base = b * (A * N)
        for a in range(A):
            def mke(j, _, _a=a):
                sv = sidx[pl.ds(j * 16, 16)]
                eidx[pl.ds(j * 16, 16)] = sv + (base + _a * N)
                return 0
            lax.fori_loop(0, 7, mke, 0)
            pltpu.async_copy(boxes_hbm.at[eidx], colbuf, sem).wait()
            for j in range(7):
                v = colbuf[pl.ds(j * 16, 16)]
                pos = (j * 16 + lanes) * A + a
                plsc.store_scatter(pbuf, [pos], v, mask=pos < K * A)

        pltpu.sync_copy(pbuf, prop_hbm.at[b])
        pltpu.sync_copy(sscore, pscore_hbm.at[b])


_MESH = plsc.VectorSubcoreMesh(core_axis_name="c", subcore_axis_name="s",
                               num_cores=2, num_subcores=16)

_SCRATCH = [
    pltpu.VMEM((N,), jnp.float32),
    pltpu.VMEM((4096,), jnp.int32),
    pltpu.VMEM((128,), jnp.int32),
    pltpu.VMEM((128,), jnp.int32),
    pltpu.VMEM((128,), jnp.float32),
    pltpu.VMEM((KPAD,), jnp.int32),
    pltpu.VMEM((SROW,), jnp.float32),
    pltpu.VMEM((KPAD,), jnp.int32),
    pltpu.VMEM((KPAD,), jnp.float32),
    pltpu.VMEM((PROW,), jnp.float32),
    pltpu.SemaphoreType.DMA,
]

_OUT = (jax.ShapeDtypeStruct((NB, PROW), jnp.float32),
        jax.ShapeDtypeStruct((NB, SROW), jnp.float32))

_topk_call = pl.kernel(_body, out_type=_OUT, mesh=_MESH,
                       scratch_types=_SCRATCH,
                       compiler_params=pltpu.CompilerParams(
                           needs_layout_passes=False))


@jax.jit
def kernel(scores, boxes):
    scores2d = scores.reshape(NB, -1)
    boxes1d = boxes.reshape(-1)
    prop_pad, ps_pad = _topk_call(scores2d, boxes1d)
    proposals = prop_pad[:, :K * A].reshape(NB, K, A)
    pscores = ps_pad[:, :K]
    return (proposals, pscores)


# speedup vs baseline: 3.2048x; 1.6267x over previous
import jax
import jax.numpy as jnp
from jax import lax
from jax.experimental import pallas as pl
from jax.experimental.pallas import tpu as pltpu
from jax.experimental.pallas import tpu_sc as plsc

NB = 8
N = 20000
NV = N // 16
K = 100
A = 7
KPAD = 112
PROW = 704
SROW = 112

SIGN = -2147483648
M31 = 2147483647


def _body(scores_hbm, boxes_hbm, prop_hbm, pscore_hbm,
          sbuf, hist, cand_k, cand_i, ckey, cidx, sidx, skey, sscore,
          eidx, colbuf, pbuf, sem):
    wid = lax.axis_index("s") * 2 + lax.axis_index("c")

    @pl.when(wid < NB)
    def _():
        b = wid
        pltpu.sync_copy(scores_hbm.at[b], sbuf)
        lanes = lax.iota(jnp.int32, 16)
        lanes256 = lanes * 256
        ones = jnp.ones((16,), jnp.int32)
        tmask = jnp.ones((16,), jnp.bool_)

        def keys_at(i):
            v = sbuf[pl.ds(i * 16, 16)]
            bits = lax.bitcast_convert_type(v, jnp.int32)
            sg = lax.shift_right_arithmetic(bits, 31)
            return bits ^ (sg | SIGN)

        def zero_hist(j, _):
            hist[pl.ds(j * 16, 16)] = jnp.zeros((16,), jnp.int32)
            return 0

        def binscan(nd):
            def chunk(t, carry):
                found, nd, dstar, base = carry
                c = 15 - t
                v = hist[pl.ds(c * 16, 16)]
                for l in range(1, 16):
                    v = v + hist[pl.ds(l * 256 + c * 16, 16)]
                suf = lax.rev(plsc.cumsum(lax.rev(v, (0,))), (0,))
                sufi = suf + base
                cond = sufi >= nd
                pc = jnp.sum(cond.astype(jnp.int32))
                lane_star = pc - 1
                cnt_at = jnp.sum(jnp.where(lanes == lane_star, v, 0))
                suf_at = jnp.sum(jnp.where(lanes == lane_star, sufi, 0))
                take = jnp.logical_and(jnp.logical_not(found), pc > 0)
                dstar = jnp.where(take, c * 16 + lane_star, dstar)
                nd = jnp.where(take, nd - (suf_at - cnt_at), nd)
                base = base + suf[0]
                return (jnp.logical_or(found, take), nd, dstar, base)
            _, nd, dstar, _ = lax.fori_loop(
                0, 16, chunk, (jnp.bool_(False), nd, jnp.int32(0),
                               jnp.int32(0)))
            return dstar, nd

        lax.fori_loop(0, 256, zero_hist, 0, unroll=8)

        def scan1(i, _):
            ku = keys_at(i)
            digit = lax.shift_right_logical(ku, 24)
            plsc.addupdate_scatter(hist, [lanes256 + digit], ones,
                                   mask=tmask)
            return 0
        lax.fori_loop(0, NV, scan1, 0, unroll=5)

        d1, need = binscan(jnp.int32(K))

        def comp_cand(i, nsel):
            ku = keys_at(i)
            m = lax.shift_right_logical(ku, 24) >= d1
            idxv = i * 16 + lanes
            plsc.store_compressed(cand_k.at[pl.ds(nsel, 16)], ku ^ SIGN,
                                  mask=m)
            plsc.store_compressed(cand_i.at[pl.ds(nsel, 16)], idxv, mask=m)
            return nsel + jnp.sum(m.astype(jnp.int32))
        ncand = lax.fori_loop(0, NV, comp_cand, jnp.int32(0), unroll=5)
        nvc = lax.shift_right_logical(ncand + 15, 4)

        pv = d1
        for p in range(2, 5):
            shift = 32 - 8 * p
            lax.fori_loop(0, 256, zero_hist, 0, unroll=8)

            def scanp(i, _, _shift=shift, _pv=pv):
                ks = cand_k[pl.ds(i * 16, 16)]
                ku = ks ^ SIGN
                digit = lax.shift_right_logical(ku, _shift) & 255
                m = jnp.logical_and(
                    lax.shift_right_logical(ku, _shift + 8) == _pv,
                    i * 16 + lanes < ncand)
                plsc.addupdate_scatter(hist, [lanes256 + digit], ones,
                                       mask=m)
                return 0
            lax.fori_loop(0, nvc, scanp, 0)

            digit, need = binscan(need)
            pv = pv * 256 + digit

        t_key = pv ^ SIGN
        quota_eq = need

        def zcand(j, _):
            ckey[pl.ds(j * 16, 16)] = jnp.full((16,), SIGN, jnp.int32)
            return 0
        lax.fori_loop(0, 8, zcand, 0)

        def zsidx(j, _):
            sidx[pl.ds(j * 16, 16)] = jnp.zeros((16,), jnp.int32)
            return 0
        lax.fori_loop(0, 7, zsidx, 0)

        def select(i, carry):
            nsel, eq_taken = carry
            ks = cand_k[pl.ds(i * 16, 16)]
            valid = i * 16 + lanes < ncand
            m_gt = jnp.logical_and(ks > t_key, valid)
            m_eq = jnp.logical_and(ks == t_key, valid)
            eq_pref = plsc.cumsum(m_eq.astype(jnp.int32))
            take_eq = jnp.logical_and(m_eq, (eq_taken + eq_pref) <= quota_eq)
            m = jnp.logical_or(m_gt, take_eq)
            iv = cand_i[pl.ds(i * 16, 16)]
            plsc.store_compressed(ckey.at[pl.ds(nsel, 16)], ks, mask=m)
            plsc.store_compressed(cidx.at[pl.ds(nsel, 16)], iv, mask=m)
            cnt = jnp.sum(m.astype(jnp.int32))
            neq = jnp.sum(take_eq.astype(jnp.int32))
            return (nsel + cnt, eq_taken + neq)
        lax.fori_loop(0, nvc, select, (jnp.int32(0), jnp.int32(0)))

        lane0 = lanes == 0

        def rankloop(e, _):
            ke = ckey[pl.ds(e, 16)][0]
            kv = ckey[pl.ds(e, 16)]
            iv = cidx[pl.ds(e, 16)]
            cnt = jnp.int32(0)
            for j in range(7):
                kj = ckey[pl.ds(j * 16, 16)]
                pj = j * 16 + lanes
                gt = kj > ke
                eq = jnp.logical_and(kj == ke, pj < e)
                cnt = cnt + jnp.sum(jnp.logical_or(gt, eq).astype(jnp.int32))
            rankv = jnp.zeros((16,), jnp.int32) + cnt
            plsc.store_scatter(skey, [rankv], kv, mask=lane0)
            plsc.store_scatter(sidx, [rankv], iv, mask=lane0)
            return 0
        lax.fori_loop(0, K, rankloop, 0)

        for j in range(7):
            ksv = skey[pl.ds(j * 16, 16)]
            sr = lax.shift_right_arithmetic(ksv, 31)
            bits = ksv ^ (sr & M31)
            sscore[pl.ds(j * 16, 16)] = lax.bitcast_convert_type(
                bits, jnp.float32)

        base = b * (A * N)
        for a in range(A):
            def mke(j, _, _a=a):
                sv = sidx[pl.ds(j * 16, 16)]
                eidx[pl.ds(j * 16, 16)] = sv + (base + _a * N)
                return 0
            lax.fori_loop(0, 7, mke, 0)
            pltpu.async_copy(boxes_hbm.at[eidx], colbuf, sem).wait()
            for j in range(7):
                v = colbuf[pl.ds(j * 16, 16)]
                pos = (j * 16 + lanes) * A + a
                plsc.store_scatter(pbuf, [pos], v, mask=pos < K * A)

        pltpu.sync_copy(pbuf, prop_hbm.at[b])
        pltpu.sync_copy(sscore, pscore_hbm.at[b])


_MESH = plsc.VectorSubcoreMesh(core_axis_name="c", subcore_axis_name="s",
                               num_cores=2, num_subcores=16)

_SCRATCH = [
    pltpu.VMEM((N,), jnp.float32),
    pltpu.VMEM((4096,), jnp.int32),
    pltpu.VMEM((N + 32,), jnp.int32),
    pltpu.VMEM((N + 32,), jnp.int32),
    pltpu.VMEM((128,), jnp.int32),
    pltpu.VMEM((128,), jnp.int32),
    pltpu.VMEM((KPAD,), jnp.int32),
    pltpu.VMEM((KPAD,), jnp.int32),
    pltpu.VMEM((SROW,), jnp.float32),
    pltpu.VMEM((KPAD,), jnp.int32),
    pltpu.VMEM((KPAD,), jnp.float32),
    pltpu.VMEM((PROW,), jnp.float32),
    pltpu.SemaphoreType.DMA,
]

_OUT = (jax.ShapeDtypeStruct((NB, PROW), jnp.float32),
        jax.ShapeDtypeStruct((NB, SROW), jnp.float32))

_topk_call = pl.kernel(_body, out_type=_OUT, mesh=_MESH,
                       scratch_types=_SCRATCH,
                       compiler_params=pltpu.CompilerParams(
                           needs_layout_passes=False))


@jax.jit
def kernel(scores, boxes):
    scores2d = scores.reshape(NB, -1)
    boxes1d = boxes.reshape(-1)
    prop_pad, ps_pad = _topk_call(scores2d, boxes1d)
    proposals = prop_pad[:, :K * A].reshape(NB, K, A)
    pscores = ps_pad[:, :K]
    return (proposals, pscores)


# speedup vs baseline: 3.3149x; 1.0344x over previous
import jax
import jax.numpy as jnp
from jax import lax
from jax.experimental import pallas as pl
from jax.experimental.pallas import tpu as pltpu
from jax.experimental.pallas import tpu_sc as plsc

NB = 8
N = 20000
NV = N // 16
K = 100
A = 7
KPAD = 112
PROW = 704
SROW = 112

SIGN = -2147483648
M31 = 2147483647


def _body(scores_hbm, boxes_hbm, prop_hbm, pscore_hbm,
          sbuf, hist, cand_k, cand_i, ckey, cidx, sidx, skey, sscore,
          eidx, colbuf, pbuf, sem):
    wid = lax.axis_index("s") * 2 + lax.axis_index("c")

    @pl.when(wid < NB)
    def _():
        b = wid
        pltpu.sync_copy(scores_hbm.at[b], sbuf)
        lanes = lax.iota(jnp.int32, 16)
        lanes256 = lanes * 256
        ones = jnp.ones((16,), jnp.int32)
        tmask = jnp.ones((16,), jnp.bool_)

        def popcnt(m):
            return plsc.all_reduce_population_count(m)[0]

        def keys_at(i):
            v = sbuf[pl.ds(i * 16, 16)]
            bits = lax.bitcast_convert_type(v, jnp.int32)
            sg = lax.shift_right_arithmetic(bits, 31)
            return bits ^ (sg | SIGN)

        def zero_hist(j, _):
            hist[pl.ds(j * 16, 16)] = jnp.zeros((16,), jnp.int32)
            return 0

        def binscan(nd):
            def chunk(t, carry):
                found, nd, dstar, base = carry
                c = 15 - t
                v = hist[pl.ds(c * 16, 16)]
                for l in range(1, 16):
                    v = v + hist[pl.ds(l * 256 + c * 16, 16)]
                suf = lax.rev(plsc.cumsum(lax.rev(v, (0,))), (0,))
                sufi = suf + base
                cond = sufi >= nd
                pc = popcnt(cond)
                lane_star = pc - 1
                cnt_at = jnp.sum(jnp.where(lanes == lane_star, v, 0))
                suf_at = jnp.sum(jnp.where(lanes == lane_star, sufi, 0))
                take = jnp.logical_and(jnp.logical_not(found), pc > 0)
                dstar = jnp.where(take, c * 16 + lane_star, dstar)
                nd = jnp.where(take, nd - (suf_at - cnt_at), nd)
                base = base + suf[0]
                return (jnp.logical_or(found, take), nd, dstar, base)
            _, nd, dstar, _ = lax.fori_loop(
                0, 16, chunk, (jnp.bool_(False), nd, jnp.int32(0),
                               jnp.int32(0)))
            return dstar, nd

        lax.fori_loop(0, 256, zero_hist, 0, unroll=8)

        def scan1(i, _):
            ku = keys_at(i)
            digit = lax.shift_right_logical(ku, 24)
            plsc.addupdate_scatter(hist, [lanes256 + digit], ones,
                                   mask=tmask)
            return 0
        lax.fori_loop(0, NV, scan1, 0, unroll=5)

        d1, need = binscan(jnp.int32(K))

        def comp_cand(i, nsel):
            ku = keys_at(i)
            m = lax.shift_right_logical(ku, 24) >= d1
            idxv = i * 16 + lanes
            plsc.store_compressed(cand_k.at[pl.ds(nsel, 16)], ku ^ SIGN,
                                  mask=m)
            plsc.store_compressed(cand_i.at[pl.ds(nsel, 16)], idxv, mask=m)
            return nsel + popcnt(m)
        ncand = lax.fori_loop(0, NV, comp_cand, jnp.int32(0), unroll=5)
        nvc = lax.shift_right_logical(ncand + 15, 4)

        pv = d1
        for p in range(2, 5):
            shift = 32 - 8 * p
            lax.fori_loop(0, 256, zero_hist, 0, unroll=8)

            def scanp(i, _, _shift=shift, _pv=pv):
                ks = cand_k[pl.ds(i * 16, 16)]
                ku = ks ^ SIGN
                digit = lax.shift_right_logical(ku, _shift) & 255
                m = jnp.logical_and(
                    lax.shift_right_logical(ku, _shift + 8) == _pv,
                    i * 16 + lanes < ncand)
                plsc.addupdate_scatter(hist, [lanes256 + digit], ones,
                                       mask=m)
                return 0
            lax.fori_loop(0, nvc, scanp, 0)

            digit, need = binscan(need)
            pv = pv * 256 + digit

        t_key = pv ^ SIGN
        quota_eq = need

        def zcand(j, _):
            ckey[pl.ds(j * 16, 16)] = jnp.full((16,), SIGN, jnp.int32)
            return 0
        lax.fori_loop(0, 8, zcand, 0)

        def zsidx(j, _):
            sidx[pl.ds(j * 16, 16)] = jnp.zeros((16,), jnp.int32)
            return 0
        lax.fori_loop(0, 7, zsidx, 0)

        def select(i, carry):
            nsel, eq_taken = carry
            ks = cand_k[pl.ds(i * 16, 16)]
            valid = i * 16 + lanes < ncand
            m_gt = jnp.logical_and(ks > t_key, valid)
            m_eq = jnp.logical_and(ks == t_key, valid)
            eq_pref = plsc.cumsum(m_eq.astype(jnp.int32))
            take_eq = jnp.logical_and(m_eq, (eq_taken + eq_pref) <= quota_eq)
            m = jnp.logical_or(m_gt, take_eq)
            iv = cand_i[pl.ds(i * 16, 16)]
            plsc.store_compressed(ckey.at[pl.ds(nsel, 16)], ks, mask=m)
            plsc.store_compressed(cidx.at[pl.ds(nsel, 16)], iv, mask=m)
            return (nsel + popcnt(m), eq_taken + popcnt(take_eq))
        lax.fori_loop(0, nvc, select, (jnp.int32(0), jnp.int32(0)))

        lane0 = lanes == 0

        def rankloop(e, _):
            kv = ckey[pl.ds(e, 16)]
            ke = kv[0]
            iv = cidx[pl.ds(e, 16)]
            cnt = jnp.int32(0)
            for j in range(7):
                kj = ckey[pl.ds(j * 16, 16)]
                pj = j * 16 + lanes
                gt = kj > ke
                eq = jnp.logical_and(kj == ke, pj < e)
                cnt = cnt + popcnt(jnp.logical_or(gt, eq))
            rankv = jnp.zeros((16,), jnp.int32) + cnt
            plsc.store_scatter(skey, [rankv], kv, mask=lane0)
            plsc.store_scatter(sidx, [rankv], iv, mask=lane0)
            return 0
        lax.fori_loop(0, K, rankloop, 0)

        for j in range(7):
            ksv = skey[pl.ds(j * 16, 16)]
            sr = lax.shift_right_arithmetic(ksv, 31)
            bits = ksv ^ (sr & M31)
            sscore[pl.ds(j * 16, 16)] = lax.bitcast_convert_type(
                bits, jnp.float32)

        base = b * (A * N)
        for a in range(A):
            def mke(j, _, _a=a):
                sv = sidx[pl.ds(j * 16, 16)]
                eidx[pl.ds(j * 16, 16)] = sv + (base + _a * N)
                return 0
            lax.fori_loop(0, 7, mke, 0)
            pltpu.async_copy(boxes_hbm.at[eidx], colbuf, sem).wait()
            for j in range(7):
                v = colbuf[pl.ds(j * 16, 16)]
                pos = (j * 16 + lanes) * A + a
                plsc.store_scatter(pbuf, [pos], v, mask=pos < K * A)

        pltpu.sync_copy(pbuf, prop_hbm.at[b])
        pltpu.sync_copy(sscore, pscore_hbm.at[b])


_MESH = plsc.VectorSubcoreMesh(core_axis_name="c", subcore_axis_name="s",
                               num_cores=2, num_subcores=16)

_SCRATCH = [
    pltpu.VMEM((N,), jnp.float32),
    pltpu.VMEM((4096,), jnp.int32),
    pltpu.VMEM((N + 32,), jnp.int32),
    pltpu.VMEM((N + 32,), jnp.int32),
    pltpu.VMEM((128,), jnp.int32),
    pltpu.VMEM((128,), jnp.int32),
    pltpu.VMEM((KPAD,), jnp.int32),
    pltpu.VMEM((KPAD,), jnp.int32),
    pltpu.VMEM((SROW,), jnp.float32),
    pltpu.VMEM((KPAD,), jnp.int32),
    pltpu.VMEM((KPAD,), jnp.float32),
    pltpu.VMEM((PROW,), jnp.float32),
    pltpu.SemaphoreType.DMA,
]

_OUT = (jax.ShapeDtypeStruct((NB, PROW), jnp.float32),
        jax.ShapeDtypeStruct((NB, SROW), jnp.float32))

_topk_call = pl.kernel(_body, out_type=_OUT, mesh=_MESH,
                       scratch_types=_SCRATCH,
                       compiler_params=pltpu.CompilerParams(
                           needs_layout_passes=False))


@jax.jit
def kernel(scores, boxes):
    scores2d = scores.reshape(NB, -1)
    boxes1d = boxes.reshape(-1)
    prop_pad, ps_pad = _topk_call(scores2d, boxes1d)
    proposals = prop_pad[:, :K * A].reshape(NB, K, A)
    pscores = ps_pad[:, :K]
    return (proposals, pscores)


# speedup vs baseline: 4.4156x; 1.3321x over previous
import jax
import jax.numpy as jnp
from jax import lax
from jax.experimental import pallas as pl
from jax.experimental.pallas import tpu as pltpu
from jax.experimental.pallas import tpu_sc as plsc

NB = 8
N = 20000
NV = N // 16
K = 100
A = 7
KPAD = 112
PROW = 704
SROW = 112

SIGN = -2147483648
M31 = 2147483647


def _body(scores_hbm, boxes_hbm, prop_hbm, pscore_hbm,
          sbuf, hist, cand_k, cand_i, ckey, cidx, sidx, skey, sscore,
          eidx, colbuf, pbuf, sem):
    wid = lax.axis_index("s") * 2 + lax.axis_index("c")

    @pl.when(wid < NB)
    def _():
        b = wid
        pltpu.sync_copy(scores_hbm.at[b], sbuf)
        lanes = lax.iota(jnp.int32, 16)
        lanes256 = lanes * 256
        ones = jnp.ones((16,), jnp.int32)
        tmask = jnp.ones((16,), jnp.bool_)

        def popcnt(m):
            return plsc.all_reduce_population_count(m)[0]

        def keys_at(i):
            v = sbuf[pl.ds(i * 16, 16)]
            bits = lax.bitcast_convert_type(v, jnp.int32)
            sg = lax.shift_right_arithmetic(bits, 31)
            return bits ^ (sg | SIGN)

        def zero_hist(j, _):
            hist[pl.ds(j * 16, 16)] = jnp.zeros((16,), jnp.int32)
            return 0

        def binscan(nd):
            def chunk(t, carry):
                found, nd, dstar, base = carry
                c = 15 - t
                v = hist[pl.ds(c * 16, 16)]
                for l in range(1, 16):
                    v = v + hist[pl.ds(l * 256 + c * 16, 16)]
                suf = lax.rev(plsc.cumsum(lax.rev(v, (0,))), (0,))
                sufi = suf + base
                cond = sufi >= nd
                pc = popcnt(cond)
                lane_star = pc - 1
                cnt_at = jnp.sum(jnp.where(lanes == lane_star, v, 0))
                suf_at = jnp.sum(jnp.where(lanes == lane_star, sufi, 0))
                take = jnp.logical_and(jnp.logical_not(found), pc > 0)
                dstar = jnp.where(take, c * 16 + lane_star, dstar)
                nd = jnp.where(take, nd - (suf_at - cnt_at), nd)
                base = base + suf[0]
                return (jnp.logical_or(found, take), nd, dstar, base)
            _, nd, dstar, _ = lax.fori_loop(
                0, 16, chunk, (jnp.bool_(False), nd, jnp.int32(0),
                               jnp.int32(0)))
            return dstar, nd

        lax.fori_loop(0, 256, zero_hist, 0, unroll=8)

        @plsc.parallel_loop(0, NV, unroll=5)
        def scan1(i):
            ku = keys_at(i)
            digit = lax.shift_right_logical(ku, 24)
            plsc.addupdate_scatter(hist, [lanes256 + digit], ones,
                                   mask=tmask)

        d1, need = binscan(jnp.int32(K))

        @plsc.parallel_loop(0, NV, unroll=5, carry=jnp.int32(0))
        def comp_cand(i, nsel):
            ku = keys_at(i)
            m = lax.shift_right_logical(ku, 24) >= d1
            idxv = i * 16 + lanes
            plsc.store_compressed(cand_k.at[pl.ds(nsel, 16)], ku ^ SIGN,
                                  mask=m)
            plsc.store_compressed(cand_i.at[pl.ds(nsel, 16)], idxv, mask=m)
            return nsel + popcnt(m)
        ncand = comp_cand
        nvc = lax.shift_right_logical(ncand + 15, 4)

        pv = d1
        for p in range(2, 5):
            shift = 32 - 8 * p
            lax.fori_loop(0, 256, zero_hist, 0, unroll=8)

            def scanp(i, _, _shift=shift, _pv=pv):
                ks = cand_k[pl.ds(i * 16, 16)]
                ku = ks ^ SIGN
                digit = lax.shift_right_logical(ku, _shift) & 255
                m = jnp.logical_and(
                    lax.shift_right_logical(ku, _shift + 8) == _pv,
                    i * 16 + lanes < ncand)
                plsc.addupdate_scatter(hist, [lanes256 + digit], ones,
                                       mask=m)
                return 0
            lax.fori_loop(0, nvc, scanp, 0)

            digit, need = binscan(need)
            pv = pv * 256 + digit

        t_key = pv ^ SIGN
        quota_eq = need

        def zcand(j, _):
            ckey[pl.ds(j * 16, 16)] = jnp.full((16,), SIGN, jnp.int32)
            return 0
        lax.fori_loop(0, 8, zcand, 0)

        def zsidx(j, _):
            sidx[pl.ds(j * 16, 16)] = jnp.zeros((16,), jnp.int32)
            return 0
        lax.fori_loop(0, 7, zsidx, 0)

        def select(i, carry):
            nsel, eq_taken = carry
            ks = cand_k[pl.ds(i * 16, 16)]
            valid = i * 16 + lanes < ncand
            m_gt = jnp.logical_and(ks > t_key, valid)
            m_eq = jnp.logical_and(ks == t_key, valid)
            eq_pref = plsc.cumsum(m_eq.astype(jnp.int32))
            take_eq = jnp.logical_and(m_eq, (eq_taken + eq_pref) <= quota_eq)
            m = jnp.logical_or(m_gt, take_eq)
            iv = cand_i[pl.ds(i * 16, 16)]
            plsc.store_compressed(ckey.at[pl.ds(nsel, 16)], ks, mask=m)
            plsc.store_compressed(cidx.at[pl.ds(nsel, 16)], iv, mask=m)
            return (nsel + popcnt(m), eq_taken + popcnt(take_eq))
        lax.fori_loop(0, nvc, select, (jnp.int32(0), jnp.int32(0)))

        lane0 = lanes == 0

        def rankloop(e, _):
            kv = ckey[pl.ds(e, 16)]
            ke = kv[0]
            iv = cidx[pl.ds(e, 16)]
            cnt = jnp.int32(0)
            for j in range(7):
                kj = ckey[pl.ds(j * 16, 16)]
                pj = j * 16 + lanes
                gt = kj > ke
                eq = jnp.logical_and(kj == ke, pj < e)
                cnt = cnt + popcnt(jnp.logical_or(gt, eq))
            rankv = jnp.zeros((16,), jnp.int32) + cnt
            plsc.store_scatter(skey, [rankv], kv, mask=lane0)
            plsc.store_scatter(sidx, [rankv], iv, mask=lane0)
            return 0
        lax.fori_loop(0, K, rankloop, 0)

        for j in range(7):
            ksv = skey[pl.ds(j * 16, 16)]
            sr = lax.shift_right_arithmetic(ksv, 31)
            bits = ksv ^ (sr & M31)
            sscore[pl.ds(j * 16, 16)] = lax.bitcast_convert_type(
                bits, jnp.float32)

        base = b * (A * N)
        for a in range(A):
            def mke(j, _, _a=a):
                sv = sidx[pl.ds(j * 16, 16)]
                eidx[pl.ds(j * 16, 16)] = sv + (base + _a * N)
                return 0
            lax.fori_loop(0, 7, mke, 0)
            pltpu.async_copy(boxes_hbm.at[eidx], colbuf, sem).wait()
            for j in range(7):
                v = colbuf[pl.ds(j * 16, 16)]
                pos = (j * 16 + lanes) * A + a
                plsc.store_scatter(pbuf, [pos], v, mask=pos < K * A)

        pltpu.sync_copy(pbuf, prop_hbm.at[b])
        pltpu.sync_copy(sscore, pscore_hbm.at[b])


_MESH = plsc.VectorSubcoreMesh(core_axis_name="c", subcore_axis_name="s",
                               num_cores=2, num_subcores=16)

_SCRATCH = [
    pltpu.VMEM((N,), jnp.float32),
    pltpu.VMEM((4096,), jnp.int32),
    pltpu.VMEM((N + 32,), jnp.int32),
    pltpu.VMEM((N + 32,), jnp.int32),
    pltpu.VMEM((128,), jnp.int32),
    pltpu.VMEM((128,), jnp.int32),
    pltpu.VMEM((KPAD,), jnp.int32),
    pltpu.VMEM((KPAD,), jnp.int32),
    pltpu.VMEM((SROW,), jnp.float32),
    pltpu.VMEM((KPAD,), jnp.int32),
    pltpu.VMEM((KPAD,), jnp.float32),
    pltpu.VMEM((PROW,), jnp.float32),
    pltpu.SemaphoreType.DMA,
]

_OUT = (jax.ShapeDtypeStruct((NB, PROW), jnp.float32),
        jax.ShapeDtypeStruct((NB, SROW), jnp.float32))

_topk_call = pl.kernel(_body, out_type=_OUT, mesh=_MESH,
                       scratch_types=_SCRATCH,
                       compiler_params=pltpu.CompilerParams(
                           needs_layout_passes=False))


@jax.jit
def kernel(scores, boxes):
    scores2d = scores.reshape(NB, -1)
    boxes1d = boxes.reshape(-1)
    prop_pad, ps_pad = _topk_call(scores2d, boxes1d)
    proposals = prop_pad[:, :K * A].reshape(NB, K, A)
    pscores = ps_pad[:, :K]
    return (proposals, pscores)


# speedup vs baseline: 4.6701x; 1.0576x over previous
import jax
import jax.numpy as jnp
from jax import lax
from jax.experimental import pallas as pl
from jax.experimental.pallas import tpu as pltpu
from jax.experimental.pallas import tpu_sc as plsc

NB = 8
N = 20000
NPAD = 20480
CH = 5120
NVCH = CH // 16
K = 100
A = 7
KPAD = 112
PROW = 704
SROW = 112
CROW = 5136
CAP = 1024

SIGN = -2147483648
M31 = 2147483647


def _body(scores_hbm, boxes_hbm, prop_hbm, pscore_hbm,
          sbuf, hist, tot, hist4, cand_k, cand_i, cnt_local, cntbuf,
          cank4, cani4, ckey, cidx, sidx, skey, sscore, eidx, colbuf,
          pbuf, sh_hist, sh_cnt, sh_ck, sh_ci, sem):
    c = lax.axis_index("c")
    s = lax.axis_index("s")
    b = c * 4 + lax.shift_right_logical(s, 2)
    m = s & 3
    g0 = lax.shift_right_logical(s, 2) * 4

    lanes = lax.iota(jnp.int32, 16)
    lanes256 = lanes * 256
    ones = jnp.ones((16,), jnp.int32)
    tmask = jnp.ones((16,), jnp.bool_)

    def popcnt(msk):
        return plsc.all_reduce_population_count(msk)[0]

    def keys_at(i):
        v = sbuf[pl.ds(i * 16, 16)]
        bits = lax.bitcast_convert_type(v, jnp.int32)
        sg = lax.shift_right_arithmetic(bits, 31)
        return bits ^ (sg | SIGN)

    def zero_hist(j, _):
        hist[pl.ds(j * 16, 16)] = jnp.zeros((16,), jnp.int32)
        return 0

    def suffix_search(load_bin, nd):
        def chunk(t, carry):
            found, nd, dstar, base = carry
            cc = 15 - t
            v = load_bin(cc)
            suf = lax.rev(plsc.cumsum(lax.rev(v, (0,))), (0,))
            sufi = suf + base
            cond = sufi >= nd
            pc = popcnt(cond)
            lane_star = pc - 1
            cnt_at = jnp.sum(jnp.where(lanes == lane_star, v, 0))
            suf_at = jnp.sum(jnp.where(lanes == lane_star, sufi, 0))
            take = jnp.logical_and(jnp.logical_not(found), pc > 0)
            dstar = jnp.where(take, cc * 16 + lane_star, dstar)
            nd = jnp.where(take, nd - (suf_at - cnt_at), nd)
            base = base + suf[0]
            return (jnp.logical_or(found, take), nd, dstar, base)
        _, nd, dstar, _ = lax.fori_loop(
            0, 16, chunk, (jnp.bool_(False), nd, jnp.int32(0), jnp.int32(0)))
        return dstar, nd

    pltpu.sync_copy(scores_hbm.at[pl.ds(b * NPAD + m * CH, CH)], sbuf)
    lax.fori_loop(0, 256, zero_hist, 0, unroll=8)

    @plsc.parallel_loop(0, NVCH, unroll=5)
    def scan1(i):
        ku = keys_at(i)
        digit = lax.shift_right_logical(ku, 24)
        plsc.addupdate_scatter(hist, [lanes256 + digit], ones, mask=tmask)

    @plsc.parallel_loop(0, 16, unroll=4)
    def lred(cc):
        acc = hist[pl.ds(cc * 16, 16)]
        for l in range(1, 16):
            acc = acc + hist[pl.ds(l * 256 + cc * 16, 16)]
        tot[pl.ds(cc * 16, 16)] = acc

    pltpu.sync_copy(tot, sh_hist.at[pl.ds(s * 256, 256)])
    plsc.subcore_barrier()
    pltpu.sync_copy(sh_hist.at[pl.ds(g0 * 256, 1024)], hist4)

    def bin4(cc):
        v = hist4[pl.ds(cc * 16, 16)]
        for w in range(1, 4):
            v = v + hist4[pl.ds(w * 256 + cc * 16, 16)]
        return v
    d1, need = suffix_search(bin4, jnp.int32(K))

    @plsc.parallel_loop(0, NVCH, unroll=5, carry=jnp.int32(0))
    def comp_cand(i, nsel):
        ku = keys_at(i)
        mk = lax.shift_right_logical(ku, 24) >= d1
        idxv = m * CH + i * 16 + lanes
        plsc.store_compressed(cand_k.at[pl.ds(nsel, 16)], ku ^ SIGN, mask=mk)
        plsc.store_compressed(cand_i.at[pl.ds(nsel, 16)], idxv, mask=mk)
        return nsel + popcnt(mk)
    ncand = comp_cand

    cnt_local[...] = jnp.zeros((16,), jnp.int32) + ncand
    pltpu.sync_copy(cnt_local, sh_cnt.at[pl.ds(s * 16, 16)])
    pltpu.sync_copy(cand_k.at[pl.ds(0, CAP)], sh_ck.at[pl.ds(s * CROW, CAP)])
    pltpu.sync_copy(cand_i.at[pl.ds(0, CAP)], sh_ci.at[pl.ds(s * CROW, CAP)])

    @pl.when(ncand > CAP)
    def _():
        pltpu.sync_copy(cand_k, sh_ck.at[pl.ds(s * CROW, CROW)])
        pltpu.sync_copy(cand_i, sh_ci.at[pl.ds(s * CROW, CROW)])

    plsc.subcore_barrier()

    @pl.when(m == 0)
    def _():
        pltpu.sync_copy(sh_cnt.at[pl.ds(g0 * 16, 64)], cntbuf)
        cw = [cntbuf[pl.ds(w * 16, 16)][0] for w in range(4)]
        for w in range(4):
            pltpu.sync_copy(sh_ck.at[pl.ds((g0 + w) * CROW, CAP)],
                            cank4.at[pl.ds(w * CROW, CAP)])
            pltpu.sync_copy(sh_ci.at[pl.ds((g0 + w) * CROW, CAP)],
                            cani4.at[pl.ds(w * CROW, CAP)])

            @pl.when(cw[w] > CAP)
            def _():
                pltpu.sync_copy(sh_ck.at[pl.ds((g0 + w) * CROW, CROW)],
                                cank4.at[pl.ds(w * CROW, CROW)])
                pltpu.sync_copy(sh_ci.at[pl.ds((g0 + w) * CROW, CROW)],
                                cani4.at[pl.ds(w * CROW, CROW)])

        nvw = [lax.shift_right_logical(cwi + 15, 4) for cwi in cw]

        pv = d1
        nd = need
        for p in range(2, 5):
            shift = 32 - 8 * p
            lax.fori_loop(0, 256, zero_hist, 0, unroll=8)
            for w in range(4):
                def scanp(i, _, _w=w, _shift=shift, _pv=pv):
                    ks = cank4[pl.ds(_w * CROW + i * 16, 16)]
                    ku = ks ^ SIGN
                    digit = lax.shift_right_logical(ku, _shift) & 255
                    mk = jnp.logical_and(
                        lax.shift_right_logical(ku, _shift + 8) == _pv,
                        i * 16 + lanes < cw[_w])
                    plsc.addupdate_scatter(hist, [lanes256 + digit], ones,
                                           mask=mk)
                    return 0
                lax.fori_loop(0, nvw[w], scanp, 0)

            def bin16(cc):
                v = hist[pl.ds(cc * 16, 16)]
                for l in range(1, 16):
                    v = v + hist[pl.ds(l * 256 + cc * 16, 16)]
                return v
            digit, nd = suffix_search(bin16, nd)
            pv = pv * 256 + digit

        t_key = pv ^ SIGN
        quota_eq = nd

        def zcand(j, _):
            ckey[pl.ds(j * 16, 16)] = jnp.full((16,), SIGN, jnp.int32)
            return 0
        lax.fori_loop(0, 8, zcand, 0)

        def zsidx(j, _):
            sidx[pl.ds(j * 16, 16)] = jnp.zeros((16,), jnp.int32)
            return 0
        lax.fori_loop(0, 7, zsidx, 0)

        carry = (jnp.int32(0), jnp.int32(0))
        for w in range(4):
            def select(i, cr, _w=w):
                nsel, eq_taken = cr
                ks = cank4[pl.ds(_w * CROW + i * 16, 16)]
                valid = i * 16 + lanes < cw[_w]
                m_gt = jnp.logical_and(ks > t_key, valid)
                m_eq = jnp.logical_and(ks == t_key, valid)
                eq_pref = plsc.cumsum(m_eq.astype(jnp.int32))
                take_eq = jnp.logical_and(m_eq,
                                          (eq_taken + eq_pref) <= quota_eq)
                mk = jnp.logical_or(m_gt, take_eq)
                iv = cani4[pl.ds(_w * CROW + i * 16, 16)]
                plsc.store_compressed(ckey.at[pl.ds(nsel, 16)], ks, mask=mk)
                plsc.store_compressed(cidx.at[pl.ds(nsel, 16)], iv, mask=mk)
                return (nsel + popcnt(mk), eq_taken + popcnt(take_eq))
            carry = lax.fori_loop(0, nvw[w], select, carry)

        lane0 = lanes == 0

        def rankloop(e, _):
            kv = ckey[pl.ds(e, 16)]
            ke = kv[0]
            iv = cidx[pl.ds(e, 16)]
            cnt = jnp.int32(0)
            for j in range(7):
                kj = ckey[pl.ds(j * 16, 16)]
                pj = j * 16 + lanes
                gt = kj > ke
                eq = jnp.logical_and(kj == ke, pj < e)
                cnt = cnt + popcnt(jnp.logical_or(gt, eq))
            rankv = jnp.zeros((16,), jnp.int32) + cnt
            plsc.store_scatter(skey, [rankv], kv, mask=lane0)
            plsc.store_scatter(sidx, [rankv], iv, mask=lane0)
            return 0
        lax.fori_loop(0, K, rankloop, 0)

        for j in range(7):
            ksv = skey[pl.ds(j * 16, 16)]
            sr = lax.shift_right_arithmetic(ksv, 31)
            bits = ksv ^ (sr & M31)
            sscore[pl.ds(j * 16, 16)] = lax.bitcast_convert_type(
                bits, jnp.float32)

        base = b * (A * N)
        for a in range(A):
            def mke(j, _, _a=a):
                sv = sidx[pl.ds(j * 16, 16)]
                eidx[pl.ds(j * 16, 16)] = sv + (base + _a * N)
                return 0
            lax.fori_loop(0, 7, mke, 0)
            pltpu.async_copy(boxes_hbm.at[eidx], colbuf, sem).wait()
            for j in range(7):
                v = colbuf[pl.ds(j * 16, 16)]
                pos = (j * 16 + lanes) * A + a
                plsc.store_scatter(pbuf, [pos], v, mask=pos < K * A)

        pltpu.sync_copy(pbuf, prop_hbm.at[b])
        pltpu.sync_copy(sscore, pscore_hbm.at[b])


_MESH = plsc.VectorSubcoreMesh(core_axis_name="c", subcore_axis_name="s",
                               num_cores=2, num_subcores=16)

_SCRATCH = [
    pltpu.VMEM((CH,), jnp.float32),
    pltpu.VMEM((4096,), jnp.int32),
    pltpu.VMEM((256,), jnp.int32),
    pltpu.VMEM((1024,), jnp.int32),
    pltpu.VMEM((CROW,), jnp.int32),
    pltpu.VMEM((CROW,), jnp.int32),
    pltpu.VMEM((16,), jnp.int32),
    pltpu.VMEM((64,), jnp.int32),
    pltpu.VMEM((4 * CROW,), jnp.int32),
    pltpu.VMEM((4 * CROW,), jnp.int32),
    pltpu.VMEM((128,), jnp.int32),
    pltpu.VMEM((128,), jnp.int32),
    pltpu.VMEM((KPAD,), jnp.int32),
    pltpu.VMEM((KPAD,), jnp.int32),
    pltpu.VMEM((SROW,), jnp.float32),
    pltpu.VMEM((KPAD,), jnp.int32),
    pltpu.VMEM((KPAD,), jnp.float32),
    pltpu.VMEM((PROW,), jnp.float32),
    pltpu.VMEM_SHARED((16 * 256,), jnp.int32),
    pltpu.VMEM_SHARED((16 * 16,), jnp.int32),
    pltpu.VMEM_SHARED((16 * CROW,), jnp.int32),
    pltpu.VMEM_SHARED((16 * CROW,), jnp.int32),
    pltpu.SemaphoreType.DMA,
]

_OUT = (jax.ShapeDtypeStruct((NB, PROW), jnp.float32),
        jax.ShapeDtypeStruct((NB, SROW), jnp.float32))

_topk_call = pl.kernel(_body, out_type=_OUT, mesh=_MESH,
                       scratch_types=_SCRATCH,
                       compiler_params=pltpu.CompilerParams(
                           needs_layout_passes=False))


@jax.jit
def kernel(scores, boxes):
    scores2d = scores.reshape(NB, -1)
    scores_pad = jnp.pad(scores2d, ((0, 0), (0, NPAD - N)),
                         constant_values=float("-inf"))
    boxes1d = boxes.reshape(-1)
    prop_pad, ps_pad = _topk_call(scores_pad.reshape(-1), boxes1d)
    proposals = prop_pad[:, :K * A].reshape(NB, K, A)
    pscores = ps_pad[:, :K]
    return (proposals, pscores)


# speedup vs baseline: 5.7904x; 1.2399x over previous
import jax
import jax.numpy as jnp
from jax import lax
from jax.experimental import pallas as pl
from jax.experimental.pallas import tpu as pltpu
from jax.experimental.pallas import tpu_sc as plsc

NB = 8
N = 20000
CH = 5120
NVCH = CH // 16
LASTB = N - CH
LASTLO = 3 * CH - LASTB
K = 100
A = 7
KPAD = 112
PROW = 704
SROW = 112
CROW = 5136
CAP = 1024

SIGN = -2147483648
M31 = 2147483647


def _topk_body(scores_hbm, sidx_hbm, pscore_hbm,
               sbuf, hist, tot, hist4, cand_k, cand_i, cnt_local, cntbuf,
               cank4, cani4, ckey, cidx, sidx, skey, sscore,
               sh_hist, sh_cnt, sh_ck, sh_ci):
    c = lax.axis_index("c")
    s = lax.axis_index("s")
    b = c * 4 + lax.shift_right_logical(s, 2)
    m = s & 3
    g0 = lax.shift_right_logical(s, 2) * 4
    cbase = jnp.where(m == 3, LASTB, m * CH)
    lo = jnp.where(m == 3, LASTLO, 0)

    lanes = lax.iota(jnp.int32, 16)
    lanes256 = lanes * 256
    ones = jnp.ones((16,), jnp.int32)

    def popcnt(msk):
        return plsc.all_reduce_population_count(msk)[0]

    def keys_at(i):
        v = sbuf[pl.ds(i * 16, 16)]
        bits = lax.bitcast_convert_type(v, jnp.int32)
        sg = lax.shift_right_arithmetic(bits, 31)
        return bits ^ (sg | SIGN)

    def zero_hist(j, _):
        hist[pl.ds(j * 16, 16)] = jnp.zeros((16,), jnp.int32)
        return 0

    def suffix_search(load_bin, nd):
        def chunk(t, carry):
            found, nd, dstar, base = carry
            cc = 15 - t
            v = load_bin(cc)
            suf = lax.rev(plsc.cumsum(lax.rev(v, (0,))), (0,))
            sufi = suf + base
            cond = sufi >= nd
            pc = popcnt(cond)
            lane_star = pc - 1
            cnt_at = jnp.sum(jnp.where(lanes == lane_star, v, 0))
            suf_at = jnp.sum(jnp.where(lanes == lane_star, sufi, 0))
            take = jnp.logical_and(jnp.logical_not(found), pc > 0)
            dstar = jnp.where(take, cc * 16 + lane_star, dstar)
            nd = jnp.where(take, nd - (suf_at - cnt_at), nd)
            base = base + suf[0]
            return (jnp.logical_or(found, take), nd, dstar, base)
        _, nd, dstar, _ = lax.fori_loop(
            0, 16, chunk, (jnp.bool_(False), nd, jnp.int32(0), jnp.int32(0)))
        return dstar, nd

    pltpu.sync_copy(scores_hbm.at[pl.ds(b * N + cbase, CH)], sbuf)
    lax.fori_loop(0, 256, zero_hist, 0, unroll=8)

    @plsc.parallel_loop(0, NVCH, unroll=5)
    def scan1(i):
        ku = keys_at(i)
        digit = lax.shift_right_logical(ku, 24)
        valid = i * 16 + lanes >= lo
        plsc.addupdate_scatter(hist, [lanes256 + digit], ones, mask=valid)

    @plsc.parallel_loop(0, 16, unroll=4)
    def lred(cc):
        acc = hist[pl.ds(cc * 16, 16)]
        for l in range(1, 16):
            acc = acc + hist[pl.ds(l * 256 + cc * 16, 16)]
        tot[pl.ds(cc * 16, 16)] = acc

    pltpu.sync_copy(tot, sh_hist.at[pl.ds(s * 256, 256)])
    plsc.subcore_barrier()
    pltpu.sync_copy(sh_hist.at[pl.ds(g0 * 256, 1024)], hist4)

    def bin4(cc):
        v = hist4[pl.ds(cc * 16, 16)]
        for w in range(1, 4):
            v = v + hist4[pl.ds(w * 256 + cc * 16, 16)]
        return v
    d1, need = suffix_search(bin4, jnp.int32(K))

    @plsc.parallel_loop(0, NVCH, unroll=5, carry=jnp.int32(0))
    def comp_cand(i, nsel):
        ku = keys_at(i)
        loc = i * 16 + lanes
        mk = jnp.logical_and(lax.shift_right_logical(ku, 24) >= d1,
                             loc >= lo)
        plsc.store_compressed(cand_k.at[pl.ds(nsel, 16)], ku ^ SIGN, mask=mk)
        plsc.store_compressed(cand_i.at[pl.ds(nsel, 16)], cbase + loc,
                              mask=mk)
        return nsel + popcnt(mk)
    ncand = comp_cand

    cnt_local[...] = jnp.zeros((16,), jnp.int32) + ncand
    pltpu.sync_copy(cnt_local, sh_cnt.at[pl.ds(s * 16, 16)])
    pltpu.sync_copy(cand_k.at[pl.ds(0, CAP)], sh_ck.at[pl.ds(s * CROW, CAP)])
    pltpu.sync_copy(cand_i.at[pl.ds(0, CAP)], sh_ci.at[pl.ds(s * CROW, CAP)])

    @pl.when(ncand > CAP)
    def _():
        pltpu.sync_copy(cand_k, sh_ck.at[pl.ds(s * CROW, CROW)])
        pltpu.sync_copy(cand_i, sh_ci.at[pl.ds(s * CROW, CROW)])

    plsc.subcore_barrier()

    @pl.when(m == 0)
    def _():
        pltpu.sync_copy(sh_cnt.at[pl.ds(g0 * 16, 64)], cntbuf)
        cw = [cntbuf[pl.ds(w * 16, 16)][0] for w in range(4)]
        for w in range(4):
            pltpu.sync_copy(sh_ck.at[pl.ds((g0 + w) * CROW, CAP)],
                            cank4.at[pl.ds(w * CROW, CAP)])
            pltpu.sync_copy(sh_ci.at[pl.ds((g0 + w) * CROW, CAP)],
                            cani4.at[pl.ds(w * CROW, CAP)])

            @pl.when(cw[w] > CAP)
            def _():
                pltpu.sync_copy(sh_ck.at[pl.ds((g0 + w) * CROW, CROW)],
                                cank4.at[pl.ds(w * CROW, CROW)])
                pltpu.sync_copy(sh_ci.at[pl.ds((g0 + w) * CROW, CROW)],
                                cani4.at[pl.ds(w * CROW, CROW)])

        nvw = [lax.shift_right_logical(cwi + 15, 4) for cwi in cw]

        pv = d1
        nd = need
        for p in range(2, 5):
            shift = 32 - 8 * p
            lax.fori_loop(0, 256, zero_hist, 0, unroll=8)
            for w in range(4):
                def scanp(i, _, _w=w, _shift=shift, _pv=pv):
                    ks = cank4[pl.ds(_w * CROW + i * 16, 16)]
                    ku = ks ^ SIGN
                    digit = lax.shift_right_logical(ku, _shift) & 255
                    mk = jnp.logical_and(
                        lax.shift_right_logical(ku, _shift + 8) == _pv,
                        i * 16 + lanes < cw[_w])
                    plsc.addupdate_scatter(hist, [lanes256 + digit], ones,
                                           mask=mk)
                    return 0
                lax.fori_loop(0, nvw[w], scanp, 0)

            def bin16(cc):
                v = hist[pl.ds(cc * 16, 16)]
                for l in range(1, 16):
                    v = v + hist[pl.ds(l * 256 + cc * 16, 16)]
                return v
            digit, nd = suffix_search(bin16, nd)
            pv = pv * 256 + digit

        t_key = pv ^ SIGN
        quota_eq = nd

        def zcand(j, _):
            ckey[pl.ds(j * 16, 16)] = jnp.full((16,), SIGN, jnp.int32)
            return 0
        lax.fori_loop(0, 8, zcand, 0)

        def zsidx(j, _):
            sidx[pl.ds(j * 16, 16)] = jnp.zeros((16,), jnp.int32)
            return 0
        lax.fori_loop(0, 7, zsidx, 0)

        carry = (jnp.int32(0), jnp.int32(0))
        for w in range(4):
            def select(i, cr, _w=w):
                nsel, eq_taken = cr
                ks = cank4[pl.ds(_w * CROW + i * 16, 16)]
                valid = i * 16 + lanes < cw[_w]
                m_gt = jnp.logical_and(ks > t_key, valid)
                m_eq = jnp.logical_and(ks == t_key, valid)
                eq_pref = plsc.cumsum(m_eq.astype(jnp.int32))
                take_eq = jnp.logical_and(m_eq,
                                          (eq_taken + eq_pref) <= quota_eq)
                mk = jnp.logical_or(m_gt, take_eq)
                iv = cani4[pl.ds(_w * CROW + i * 16, 16)]
                plsc.store_compressed(ckey.at[pl.ds(nsel, 16)], ks, mask=mk)
                plsc.store_compressed(cidx.at[pl.ds(nsel, 16)], iv, mask=mk)
                return (nsel + popcnt(mk), eq_taken + popcnt(take_eq))
            carry = lax.fori_loop(0, nvw[w], select, carry)

        lane0 = lanes == 0

        def rankloop(e, _):
            kv = ckey[pl.ds(e, 16)]
            ke = kv[0]
            iv = cidx[pl.ds(e, 16)]
            cnt = jnp.int32(0)
            for j in range(7):
                kj = ckey[pl.ds(j * 16, 16)]
                pj = j * 16 + lanes
                gt = kj > ke
                eq = jnp.logical_and(kj == ke, pj < e)
                cnt = cnt + popcnt(jnp.logical_or(gt, eq))
            rankv = jnp.zeros((16,), jnp.int32) + cnt
            plsc.store_scatter(skey, [rankv], kv, mask=lane0)
            plsc.store_scatter(sidx, [rankv], iv, mask=lane0)
            return 0
        lax.fori_loop(0, K, rankloop, 0)

        for j in range(7):
            ksv = skey[pl.ds(j * 16, 16)]
            sr = lax.shift_right_arithmetic(ksv, 31)
            bits = ksv ^ (sr & M31)
            sscore[pl.ds(j * 16, 16)] = lax.bitcast_convert_type(
                bits, jnp.float32)

        pltpu.sync_copy(sidx, sidx_hbm.at[b])
        pltpu.sync_copy(sscore, pscore_hbm.at[b])


def _gather_body(boxes_hbm, sidx_hbm, prop_hbm,
                 sidxb, eidx, colbuf, pbuf, sem):
    wid = lax.axis_index("s") * 2 + lax.axis_index("c")
    lanes = lax.iota(jnp.int32, 16)

    @pl.when(wid < NB)
    def _():
        b = wid
        pltpu.sync_copy(sidx_hbm.at[b], sidxb)
        base = b * (A * N)
        for a in range(A):
            def mke(j, _, _a=a):
                sv = sidxb[pl.ds(j * 16, 16)]
                eidx[pl.ds(_a * KPAD + j * 16, 16)] = sv + (base + _a * N)
                return 0
            lax.fori_loop(0, 7, mke, 0)
        waits = []
        for a in range(A):
            waits.append(pltpu.async_copy(
                boxes_hbm.at[eidx.at[pl.ds(a * KPAD, KPAD)]],
                colbuf.at[pl.ds(a * KPAD, KPAD)], sem))
        for wt in waits:
            wt.wait()
        for a in range(A):
            for j in range(7):
                v = colbuf[pl.ds(a * KPAD + j * 16, 16)]
                pos = (j * 16 + lanes) * A + a
                plsc.store_scatter(pbuf, [pos], v, mask=pos < K * A)
        pltpu.sync_copy(pbuf, prop_hbm.at[b])


_MESH = plsc.VectorSubcoreMesh(core_axis_name="c", subcore_axis_name="s",
                               num_cores=2, num_subcores=16)

_TOPK_SCRATCH = [
    pltpu.VMEM((CH,), jnp.float32),
    pltpu.VMEM((4096,), jnp.int32),
    pltpu.VMEM((256,), jnp.int32),
    pltpu.VMEM((1024,), jnp.int32),
    pltpu.VMEM((CROW,), jnp.int32),
    pltpu.VMEM((CROW,), jnp.int32),
    pltpu.VMEM((16,), jnp.int32),
    pltpu.VMEM((64,), jnp.int32),
    pltpu.VMEM((4 * CROW,), jnp.int32),
    pltpu.VMEM((4 * CROW,), jnp.int32),
    pltpu.VMEM((128,), jnp.int32),
    pltpu.VMEM((128,), jnp.int32),
    pltpu.VMEM((SROW,), jnp.int32),
    pltpu.VMEM((SROW,), jnp.int32),
    pltpu.VMEM((SROW,), jnp.float32),
    pltpu.VMEM_SHARED((16 * 256,), jnp.int32),
    pltpu.VMEM_SHARED((16 * 16,), jnp.int32),
    pltpu.VMEM_SHARED((16 * CROW,), jnp.int32),
    pltpu.VMEM_SHARED((16 * CROW,), jnp.int32),
]

_GATHER_SCRATCH = [
    pltpu.VMEM((SROW,), jnp.int32),
    pltpu.VMEM((A * KPAD,), jnp.int32),
    pltpu.VMEM((A * KPAD,), jnp.float32),
    pltpu.VMEM((PROW,), jnp.float32),
    pltpu.SemaphoreType.DMA,
]

_topk_call = pl.kernel(
    _topk_body,
    out_type=(jax.ShapeDtypeStruct((NB, SROW), jnp.int32),
              jax.ShapeDtypeStruct((NB, SROW), jnp.float32)),
    mesh=_MESH, scratch_types=_TOPK_SCRATCH,
    compiler_params=pltpu.CompilerParams(needs_layout_passes=False))

_gather_call = pl.kernel(
    _gather_body,
    out_type=jax.ShapeDtypeStruct((NB, PROW), jnp.float32),
    mesh=_MESH, scratch_types=_GATHER_SCRATCH,
    compiler_params=pltpu.CompilerParams(needs_layout_passes=False))


@jax.jit
def kernel(scores, boxes):
    scores1d = scores.reshape(-1)
    boxes1d = boxes.reshape(-1)
    sidx_pad, ps_pad = _topk_call(scores1d)
    prop_pad = _gather_call(boxes1d, sidx_pad)
    proposals = prop_pad[:, :K * A].reshape(NB, K, A)
    pscores = ps_pad[:, :K]
    return (proposals, pscores)


# speedup vs baseline: 5.9134x; 1.0212x over previous
import jax
import jax.numpy as jnp
from jax import lax
from jax.experimental import pallas as pl
from jax.experimental.pallas import tpu as pltpu
from jax.experimental.pallas import tpu_sc as plsc

NB = 8
N = 20000
CH = 5120
NVCH = CH // 16
LASTB = N - CH
LASTLO = 3 * CH - LASTB
K = 100
A = 7
KPAD = 112
PROW = 704
SROW = 112
CROW = 5136
CAP = 1024

SIGN = -2147483648
M31 = 2147483647


def _topk_body(scores_hbm, sidx_hbm, pscore_hbm,
               sbuf, hist, tot, hist4, cand_k, cand_i, cnt_local, cntbuf,
               cank4, cani4, ckey, cidx, sidx, skey, sscore,
               sh_hist, sh_cnt, sh_ck, sh_ci):
    c = lax.axis_index("c")
    s = lax.axis_index("s")
    b = c * 4 + lax.shift_right_logical(s, 2)
    m = s & 3
    g0 = lax.shift_right_logical(s, 2) * 4
    cbase = jnp.where(m == 3, LASTB, m * CH)
    lo = jnp.where(m == 3, LASTLO, 0)

    lanes = lax.iota(jnp.int32, 16)
    lanes256 = lanes * 256
    ones = jnp.ones((16,), jnp.int32)

    def popcnt(msk):
        return plsc.all_reduce_population_count(msk)[0]

    def keys_at(i):
        v = sbuf[pl.ds(i * 16, 16)]
        bits = lax.bitcast_convert_type(v, jnp.int32)
        sg = lax.shift_right_arithmetic(bits, 31)
        return bits ^ (sg | SIGN)

    def zero_hist(j, _):
        hist[pl.ds(j * 16, 16)] = jnp.zeros((16,), jnp.int32)
        return 0

    def suffix_search(load_bin, nd):
        def chunk(t, carry):
            found, nd, dstar, base = carry
            cc = 15 - t
            v = load_bin(cc)
            suf = lax.rev(plsc.cumsum(lax.rev(v, (0,))), (0,))
            sufi = suf + base
            cond = sufi >= nd
            pc = popcnt(cond)
            lane_star = pc - 1
            cnt_at = jnp.sum(jnp.where(lanes == lane_star, v, 0))
            suf_at = jnp.sum(jnp.where(lanes == lane_star, sufi, 0))
            take = jnp.logical_and(jnp.logical_not(found), pc > 0)
            dstar = jnp.where(take, cc * 16 + lane_star, dstar)
            nd = jnp.where(take, nd - (suf_at - cnt_at), nd)
            base = base + suf[0]
            return (jnp.logical_or(found, take), nd, dstar, base)
        _, nd, dstar, _ = lax.fori_loop(
            0, 16, chunk, (jnp.bool_(False), nd, jnp.int32(0), jnp.int32(0)))
        return dstar, nd

    pltpu.sync_copy(scores_hbm.at[pl.ds(b * N + cbase, CH)], sbuf)
    lax.fori_loop(0, 256, zero_hist, 0, unroll=8)

    @plsc.parallel_loop(0, NVCH, unroll=5)
    def scan1(i):
        ku = keys_at(i)
        digit = lax.shift_right_logical(ku, 24)
        valid = i * 16 + lanes >= lo
        plsc.addupdate_scatter(hist, [lanes256 + digit], ones, mask=valid)

    @plsc.parallel_loop(0, 16, unroll=4)
    def lred(cc):
        acc = hist[pl.ds(cc * 16, 16)]
        for l in range(1, 16):
            acc = acc + hist[pl.ds(l * 256 + cc * 16, 16)]
        tot[pl.ds(cc * 16, 16)] = acc

    pltpu.sync_copy(tot, sh_hist.at[pl.ds(s * 256, 256)])
    plsc.subcore_barrier()
    pltpu.sync_copy(sh_hist.at[pl.ds(g0 * 256, 1024)], hist4)

    def bin4(cc):
        v = hist4[pl.ds(cc * 16, 16)]
        for w in range(1, 4):
            v = v + hist4[pl.ds(w * 256 + cc * 16, 16)]
        return v
    d1, need = suffix_search(bin4, jnp.int32(K))

    @plsc.parallel_loop(0, NVCH, unroll=5, carry=jnp.int32(0))
    def comp_cand(i, nsel):
        ku = keys_at(i)
        loc = i * 16 + lanes
        mk = jnp.logical_and(lax.shift_right_logical(ku, 24) >= d1,
                             loc >= lo)
        plsc.store_compressed(cand_k.at[pl.ds(nsel, 16)], ku ^ SIGN, mask=mk)
        plsc.store_compressed(cand_i.at[pl.ds(nsel, 16)], cbase + loc,
                              mask=mk)
        return nsel + popcnt(mk)
    ncand = comp_cand

    cnt_local[...] = jnp.zeros((16,), jnp.int32) + ncand
    pltpu.sync_copy(cnt_local, sh_cnt.at[pl.ds(s * 16, 16)])
    pltpu.sync_copy(cand_k.at[pl.ds(0, CAP)], sh_ck.at[pl.ds(s * CROW, CAP)])
    pltpu.sync_copy(cand_i.at[pl.ds(0, CAP)], sh_ci.at[pl.ds(s * CROW, CAP)])

    @pl.when(ncand > CAP)
    def _():
        pltpu.sync_copy(cand_k, sh_ck.at[pl.ds(s * CROW, CROW)])
        pltpu.sync_copy(cand_i, sh_ci.at[pl.ds(s * CROW, CROW)])

    plsc.subcore_barrier()

    @pl.when(m == 0)
    def _():
        pltpu.sync_copy(sh_cnt.at[pl.ds(g0 * 16, 64)], cntbuf)
        cw = [cntbuf[pl.ds(w * 16, 16)][0] for w in range(4)]
        for w in range(4):
            pltpu.sync_copy(sh_ck.at[pl.ds((g0 + w) * CROW, CAP)],
                            cank4.at[pl.ds(w * CROW, CAP)])
            pltpu.sync_copy(sh_ci.at[pl.ds((g0 + w) * CROW, CAP)],
                            cani4.at[pl.ds(w * CROW, CAP)])

            @pl.when(cw[w] > CAP)
            def _():
                pltpu.sync_copy(sh_ck.at[pl.ds((g0 + w) * CROW, CROW)],
                                cank4.at[pl.ds(w * CROW, CROW)])
                pltpu.sync_copy(sh_ci.at[pl.ds((g0 + w) * CROW, CROW)],
                                cani4.at[pl.ds(w * CROW, CROW)])

        nvw = [lax.shift_right_logical(cwi + 15, 4) for cwi in cw]

        pv = d1
        nd = need
        for p in range(2, 5):
            shift = 32 - 8 * p
            lax.fori_loop(0, 256, zero_hist, 0, unroll=8)
            for w in range(4):
                def scanp(i, _, _w=w, _shift=shift, _pv=pv):
                    ks = cank4[pl.ds(_w * CROW + i * 16, 16)]
                    ku = ks ^ SIGN
                    digit = lax.shift_right_logical(ku, _shift) & 255
                    mk = jnp.logical_and(
                        lax.shift_right_logical(ku, _shift + 8) == _pv,
                        i * 16 + lanes < cw[_w])
                    plsc.addupdate_scatter(hist, [lanes256 + digit], ones,
                                           mask=mk)
                    return 0
                lax.fori_loop(0, nvw[w], scanp, 0)

            def bin16(cc):
                v = hist[pl.ds(cc * 16, 16)]
                for l in range(1, 16):
                    v = v + hist[pl.ds(l * 256 + cc * 16, 16)]
                return v
            digit, nd = suffix_search(bin16, nd)
            pv = pv * 256 + digit

        t_key = pv ^ SIGN
        quota_eq = nd

        def zcand(j, _):
            ckey[pl.ds(j * 16, 16)] = jnp.full((16,), SIGN, jnp.int32)
            return 0
        lax.fori_loop(0, 8, zcand, 0)

        def zsidx(j, _):
            sidx[pl.ds(j * 16, 16)] = jnp.zeros((16,), jnp.int32)
            return 0
        lax.fori_loop(0, 7, zsidx, 0)

        carry = (jnp.int32(0), jnp.int32(0))
        for w in range(4):
            def select(i, cr, _w=w):
                nsel, eq_taken = cr
                ks = cank4[pl.ds(_w * CROW + i * 16, 16)]
                valid = i * 16 + lanes < cw[_w]
                m_gt = jnp.logical_and(ks > t_key, valid)
                m_eq = jnp.logical_and(ks == t_key, valid)
                eq_pref = plsc.cumsum(m_eq.astype(jnp.int32))
                take_eq = jnp.logical_and(m_eq,
                                          (eq_taken + eq_pref) <= quota_eq)
                mk = jnp.logical_or(m_gt, take_eq)
                iv = cani4[pl.ds(_w * CROW + i * 16, 16)]
                plsc.store_compressed(ckey.at[pl.ds(nsel, 16)], ks, mask=mk)
                plsc.store_compressed(cidx.at[pl.ds(nsel, 16)], iv, mask=mk)
                return (nsel + popcnt(mk), eq_taken + popcnt(take_eq))
            carry = lax.fori_loop(0, nvw[w], select, carry)

        lane0 = lanes == 0

        def rankloop(e, _):
            kv = ckey[pl.ds(e, 16)]
            ke = kv[0]
            iv = cidx[pl.ds(e, 16)]
            cnt = jnp.int32(0)
            for j in range(7):
                kj = ckey[pl.ds(j * 16, 16)]
                pj = j * 16 + lanes
                gt = kj > ke
                eq = jnp.logical_and(kj == ke, pj < e)
                cnt = cnt + popcnt(jnp.logical_or(gt, eq))
            rankv = jnp.zeros((16,), jnp.int32) + cnt
            plsc.store_scatter(skey, [rankv], kv, mask=lane0)
            plsc.store_scatter(sidx, [rankv], iv, mask=lane0)
            return 0
        lax.fori_loop(0, K, rankloop, 0, unroll=4)

        for j in range(7):
            ksv = skey[pl.ds(j * 16, 16)]
            sr = lax.shift_right_arithmetic(ksv, 31)
            bits = ksv ^ (sr & M31)
            sscore[pl.ds(j * 16, 16)] = lax.bitcast_convert_type(
                bits, jnp.float32)

        pltpu.sync_copy(sidx, sidx_hbm.at[b])
        pltpu.sync_copy(sscore, pscore_hbm.at[b])


def _gather_body(boxes_hbm, sidx_hbm, prop_hbm,
                 sidxb, eidx, colbuf, pbuf, sem):
    wid = lax.axis_index("s") * 2 + lax.axis_index("c")
    lanes = lax.iota(jnp.int32, 16)

    @pl.when(wid < NB)
    def _():
        b = wid
        pltpu.sync_copy(sidx_hbm.at[b], sidxb)
        base = b * (A * N)
        for a in range(A):
            def mke(j, _, _a=a):
                sv = sidxb[pl.ds(j * 16, 16)]
                eidx[pl.ds(_a * KPAD + j * 16, 16)] = sv + (base + _a * N)
                return 0
            lax.fori_loop(0, 7, mke, 0, unroll=7)
        waits = []
        for a in range(A):
            waits.append(pltpu.async_copy(
                boxes_hbm.at[eidx.at[pl.ds(a * KPAD, KPAD)]],
                colbuf.at[pl.ds(a * KPAD, KPAD)], sem))
        for wt in waits:
            wt.wait()
        for a in range(A):
            for j in range(7):
                v = colbuf[pl.ds(a * KPAD + j * 16, 16)]
                pos = (j * 16 + lanes) * A + a
                plsc.store_scatter(pbuf, [pos], v, mask=pos < K * A)
        pltpu.sync_copy(pbuf, prop_hbm.at[b])


_MESH = plsc.VectorSubcoreMesh(core_axis_name="c", subcore_axis_name="s",
                               num_cores=2, num_subcores=16)

_TOPK_SCRATCH = [
    pltpu.VMEM((CH,), jnp.float32),
    pltpu.VMEM((4096,), jnp.int32),
    pltpu.VMEM((256,), jnp.int32),
    pltpu.VMEM((1024,), jnp.int32),
    pltpu.VMEM((CROW,), jnp.int32),
    pltpu.VMEM((CROW,), jnp.int32),
    pltpu.VMEM((16,), jnp.int32),
    pltpu.VMEM((64,), jnp.int32),
    pltpu.VMEM((4 * CROW,), jnp.int32),
    pltpu.VMEM((4 * CROW,), jnp.int32),
    pltpu.VMEM((128,), jnp.int32),
    pltpu.VMEM((128,), jnp.int32),
    pltpu.VMEM((SROW,), jnp.int32),
    pltpu.VMEM((SROW,), jnp.int32),
    pltpu.VMEM((SROW,), jnp.float32),
    pltpu.VMEM_SHARED((16 * 256,), jnp.int32),
    pltpu.VMEM_SHARED((16 * 16,), jnp.int32),
    pltpu.VMEM_SHARED((16 * CROW,), jnp.int32),
    pltpu.VMEM_SHARED((16 * CROW,), jnp.int32),
]

_GATHER_SCRATCH = [
    pltpu.VMEM((SROW,), jnp.int32),
    pltpu.VMEM((A * KPAD,), jnp.int32),
    pltpu.VMEM((A * KPAD,), jnp.float32),
    pltpu.VMEM((PROW,), jnp.float32),
    pltpu.SemaphoreType.DMA,
]

_topk_call = pl.kernel(
    _topk_body,
    out_type=(jax.ShapeDtypeStruct((NB, SROW), jnp.int32),
              jax.ShapeDtypeStruct((NB, SROW), jnp.float32)),
    mesh=_MESH, scratch_types=_TOPK_SCRATCH,
    compiler_params=pltpu.CompilerParams(needs_layout_passes=False))

_gather_call = pl.kernel(
    _gather_body,
    out_type=jax.ShapeDtypeStruct((NB, PROW), jnp.float32),
    mesh=_MESH, scratch_types=_GATHER_SCRATCH,
    compiler_params=pltpu.CompilerParams(needs_layout_passes=False))


@jax.jit
def kernel(scores, boxes):
    scores1d = scores.reshape(-1)
    boxes1d = boxes.reshape(-1)
    sidx_pad, ps_pad = _topk_call(scores1d)
    prop_pad = _gather_call(boxes1d, sidx_pad)
    proposals = prop_pad[:, :K * A].reshape(NB, K, A)
    pscores = ps_pad[:, :K]
    return (proposals, pscores)
